# baseline (speedup 1.0000x reference)
import functools

import jax
import jax.numpy as jnp
from jax.experimental import pallas as pl
from jax.experimental.pallas import tpu as pltpu

BT = 2048


def _router_body(h_ref, w_ref, logits_ref, probs_ref, mask_ref):
    h = h_ref[...]
    w = w_ref[...]
    logits = jax.lax.dot_general(
        h, w, (((1,), (1,)), ((), ())), preferred_element_type=jnp.float32
    )
    logits_ref[...] = logits
    m = jnp.max(logits, axis=1, keepdims=True)
    ex = jnp.exp(logits - m)
    probs_ref[...] = ex / jnp.sum(ex, axis=1, keepdims=True)

    e = logits.shape[1]
    col = jax.lax.broadcasted_iota(jnp.int32, logits.shape, 1)
    cand1 = jnp.where(logits == m, col, e)
    i1 = jnp.min(cand1, axis=1, keepdims=True)
    take1 = col == i1
    v2 = jnp.where(take1, -jnp.inf, logits)
    m2 = jnp.max(v2, axis=1, keepdims=True)
    cand2 = jnp.where(v2 == m2, col, e)
    i2 = jnp.min(cand2, axis=1, keepdims=True)
    mask_ref[...] = (take1 | (col == i2)).astype(mask_ref.dtype)


@jax.jit
def kernel(h, W):
    t, d = h.shape
    e = W.shape[0]
    grid = (t // BT,)
    logits, probs, mask = pl.pallas_call(
        _router_body,
        grid=grid,
        in_specs=[
            pl.BlockSpec((BT, d), lambda i: (i, 0)),
            pl.BlockSpec((e, d), lambda i: (0, 0)),
        ],
        out_specs=[
            pl.BlockSpec((BT, e), lambda i: (i, 0)),
            pl.BlockSpec((BT, e), lambda i: (i, 0)),
            pl.BlockSpec((BT, e), lambda i: (i, 0)),
        ],
        out_shape=[
            jax.ShapeDtypeStruct((t, e), jnp.float32),
            jax.ShapeDtypeStruct((t, e), jnp.float32),
            jax.ShapeDtypeStruct((t, e), jnp.int8),
        ],
        compiler_params=pltpu.CompilerParams(
            dimension_semantics=("arbitrary",),
        ),
    )(h, W)
    return (mask.astype(bool), probs, logits, logits)


# speedup vs baseline: 2.0300x; 2.0300x over previous
import functools

import jax
import jax.numpy as jnp
from jax.experimental import pallas as pl
from jax.experimental.pallas import tpu as pltpu

BT = 2048


def _router_body(h_ref, w_ref, logits_ref, probs_ref, mask_ref):
    h = h_ref[...]
    w = w_ref[...]
    logits = jax.lax.dot_general(
        w, h, (((1,), (1,)), ((), ())), preferred_element_type=jnp.float32
    )
    logits_ref[...] = logits
    m1 = jnp.max(logits, axis=0, keepdims=True)
    ex = jnp.exp(logits - m1)
    probs_ref[...] = ex / jnp.sum(ex, axis=0, keepdims=True)

    e = logits.shape[0]
    row = jax.lax.broadcasted_iota(jnp.int32, logits.shape, 0)
    cand1 = jnp.where(logits == m1, row, e)
    i1 = jnp.min(cand1, axis=0, keepdims=True)
    take1 = row == i1
    v2 = jnp.where(take1, -jnp.inf, logits)
    m2 = jnp.max(v2, axis=0, keepdims=True)
    cand2 = jnp.where(v2 == m2, row, e)
    i2 = jnp.min(cand2, axis=0, keepdims=True)
    mask_ref[...] = (take1 | (row == i2)).astype(mask_ref.dtype)


@jax.jit
def kernel(h, W):
    t, d = h.shape
    e = W.shape[0]
    grid = (t // BT,)
    logits_t, probs_t, mask_t = pl.pallas_call(
        _router_body,
        grid=grid,
        in_specs=[
            pl.BlockSpec((BT, d), lambda i: (i, 0)),
            pl.BlockSpec((e, d), lambda i: (0, 0)),
        ],
        out_specs=[
            pl.BlockSpec((e, BT), lambda i: (0, i)),
            pl.BlockSpec((e, BT), lambda i: (0, i)),
            pl.BlockSpec((e, BT), lambda i: (0, i)),
        ],
        out_shape=[
            jax.ShapeDtypeStruct((e, t), jnp.float32),
            jax.ShapeDtypeStruct((e, t), jnp.float32),
            jax.ShapeDtypeStruct((e, t), jnp.float32),
        ],
        compiler_params=pltpu.CompilerParams(
            dimension_semantics=("arbitrary",),
        ),
    )(h, W)
    logits = logits_t.T
    return (mask_t.T.astype(bool), probs_t.T, logits, logits)
